# Initial kernel scaffold; baseline (speedup 1.0000x reference)
import jax
import jax.numpy as jnp
from jax.experimental import pallas as pl


def kernel(node_attn, edge_index):
    raise NotImplementedError("write your pallas kernel here")


# speedup vs baseline: 4.9679x; 4.9679x over previous
import functools

import jax
import jax.numpy as jnp
from jax import lax
from jax.experimental import pallas as pl
from jax.experimental.pallas import tpu as pltpu
from jax.experimental.pallas import tpu_sc as plsc

N_NODES = 10000
N_EDGES = 320000
D = 128
NW = 32
E_PER_W = N_EDGES // NW
CHUNK = 400
N_CHUNKS = E_PER_W // CHUNK


def _edge_attn_body(node_hbm, src_hbm, dst_hbm, out_hbm,
                    idx_s, idx_d, rows_s, rows_d, sem_s, sem_d):
    wid = lax.axis_index("s") * 2 + lax.axis_index("c")
    w_base = wid * E_PER_W

    def chunk_body(c, _):
        base = w_base + c * CHUNK
        pltpu.sync_copy(src_hbm.at[pl.ds(base, CHUNK)], idx_s)
        pltpu.sync_copy(dst_hbm.at[pl.ds(base, CHUNK)], idx_d)
        cp_s = pltpu.async_copy(node_hbm.at[idx_s], rows_s, sem_s)
        cp_d = pltpu.async_copy(node_hbm.at[idx_d], rows_d, sem_d)
        cp_s.wait()
        cp_d.wait()

        def mul_body(i, _):
            for j in range(D // 16):
                sl = (i, pl.ds(j * 16, 16))
                rows_s[sl] = rows_s[sl] * rows_d[sl]
            return 0

        lax.fori_loop(0, CHUNK, mul_body, 0)
        pltpu.sync_copy(rows_s, out_hbm.at[pl.ds(base, CHUNK)])
        return 0

    lax.fori_loop(0, N_CHUNKS, chunk_body, 0)


@jax.jit
def _edge_attn(node_attn, src, dst):
    mesh = plsc.VectorSubcoreMesh(core_axis_name="c", subcore_axis_name="s")
    return pl.kernel(
        _edge_attn_body,
        mesh=mesh,
        out_type=jax.ShapeDtypeStruct((N_EDGES, D), jnp.float32),
        scratch_types=[
            pltpu.VMEM((CHUNK,), jnp.int32),
            pltpu.VMEM((CHUNK,), jnp.int32),
            pltpu.VMEM((CHUNK, D), jnp.float32),
            pltpu.VMEM((CHUNK, D), jnp.float32),
            pltpu.SemaphoreType.DMA,
            pltpu.SemaphoreType.DMA,
        ],
    )(node_attn, src, dst)


def kernel(node_attn, edge_index):
    src = edge_index[0].astype(jnp.int32)
    dst = edge_index[1].astype(jnp.int32)
    return _edge_attn(node_attn, src, dst)


# speedup vs baseline: 7.6189x; 1.5336x over previous
import jax
import jax.numpy as jnp
from jax import lax
from jax.experimental import pallas as pl
from jax.experimental.pallas import tpu as pltpu
from jax.experimental.pallas import tpu_sc as plsc

N_NODES = 10000
N_EDGES = 320000
D = 128
NW = 32
E_PER_W = N_EDGES // NW
CHUNK = 200
N_CHUNKS = E_PER_W // CHUNK


def _edge_attn_body(node_hbm, src_hbm, dst_hbm, out_hbm,
                    idx_s, idx_d,
                    rows_s0, rows_d0, rows_s1, rows_d1,
                    sem_s0, sem_d0, sem_s1, sem_d1):
    wid = lax.axis_index("s") * 2 + lax.axis_index("c")
    w_base = wid * E_PER_W

    rows_s = (rows_s0, rows_s1)
    rows_d = (rows_d0, rows_d1)
    sem_s = (sem_s0, sem_s1)
    sem_d = (sem_d0, sem_d1)

    pltpu.sync_copy(src_hbm.at[pl.ds(w_base, E_PER_W)], idx_s)
    pltpu.sync_copy(dst_hbm.at[pl.ds(w_base, E_PER_W)], idx_d)

    def fire(g, b):
        pltpu.async_copy(node_hbm.at[idx_s.at[pl.ds(g * CHUNK, CHUNK)]],
                         rows_s[b], sem_s[b])
        pltpu.async_copy(node_hbm.at[idx_d.at[pl.ds(g * CHUNK, CHUNK)]],
                         rows_d[b], sem_d[b])

    def drain(b):
        dummy = out_hbm.at[pl.ds(0, CHUNK)]
        pltpu.make_async_copy(dummy, rows_s[b], sem_s[b]).wait()
        pltpu.make_async_copy(dummy, rows_d[b], sem_d[b]).wait()

    def process(g, b):
        drain(b)

        def mul_body(i, _):
            for j in range(D // 16):
                sl = (i, pl.ds(j * 16, 16))
                rows_s[b][sl] = rows_s[b][sl] * rows_d[b][sl]
            return 0

        lax.fori_loop(0, CHUNK, mul_body, 0)
        pltpu.sync_copy(rows_s[b], out_hbm.at[pl.ds(w_base + g * CHUNK, CHUNK)])

    fire(0, 0)

    def chunk_pair(g2, _):
        g = g2 * 2
        fire(g + 1, 1)
        process(g, 0)

        @pl.when(g2 < N_CHUNKS // 2 - 1)
        def _():
            fire(g + 2, 0)

        process(g + 1, 1)
        return 0

    lax.fori_loop(0, N_CHUNKS // 2, chunk_pair, 0)


@jax.jit
def _edge_attn(node_attn, src, dst):
    mesh = plsc.VectorSubcoreMesh(core_axis_name="c", subcore_axis_name="s")
    return pl.kernel(
        _edge_attn_body,
        mesh=mesh,
        out_type=jax.ShapeDtypeStruct((N_EDGES, D), jnp.float32),
        scratch_types=[
            pltpu.VMEM((E_PER_W,), jnp.int32),
            pltpu.VMEM((E_PER_W,), jnp.int32),
            pltpu.VMEM((CHUNK, D), jnp.float32),
            pltpu.VMEM((CHUNK, D), jnp.float32),
            pltpu.VMEM((CHUNK, D), jnp.float32),
            pltpu.VMEM((CHUNK, D), jnp.float32),
            pltpu.SemaphoreType.DMA,
            pltpu.SemaphoreType.DMA,
            pltpu.SemaphoreType.DMA,
            pltpu.SemaphoreType.DMA,
        ],
    )(node_attn, src, dst)


def kernel(node_attn, edge_index):
    src = edge_index[0].astype(jnp.int32)
    dst = edge_index[1].astype(jnp.int32)
    return _edge_attn(node_attn, src, dst)


# speedup vs baseline: 7.6382x; 1.0025x over previous
import jax
import jax.numpy as jnp
from jax import lax
from jax.experimental import pallas as pl
from jax.experimental.pallas import tpu as pltpu
from jax.experimental.pallas import tpu_sc as plsc

N_NODES = 10000
N_EDGES = 320000
D = 128
NW = 32
E_PER_W = N_EDGES // NW
CHUNK = 80
N_CHUNKS = E_PER_W // CHUNK
NBUF = 3


def _edge_attn_body(node_hbm, src_hbm, dst_hbm, out_hbm,
                    idx_s, idx_d,
                    rs0, rd0, rs1, rd1, rs2, rd2,
                    ss0, sd0, ss1, sd1, ss2, sd2,
                    so0, so1, so2):
    wid = lax.axis_index("s") * 2 + lax.axis_index("c")
    w_base = wid * E_PER_W

    rows_s = (rs0, rs1, rs2)
    rows_d = (rd0, rd1, rd2)
    sem_s = (ss0, ss1, ss2)
    sem_d = (sd0, sd1, sd2)
    sem_o = (so0, so1, so2)

    pltpu.sync_copy(src_hbm.at[pl.ds(w_base, E_PER_W)], idx_s)
    pltpu.sync_copy(dst_hbm.at[pl.ds(w_base, E_PER_W)], idx_d)

    def fire_gather(g, b):
        pltpu.async_copy(node_hbm.at[idx_s.at[pl.ds(g * CHUNK, CHUNK)]],
                         rows_s[b], sem_s[b])
        pltpu.async_copy(node_hbm.at[idx_d.at[pl.ds(g * CHUNK, CHUNK)]],
                         rows_d[b], sem_d[b])

    def drain_gather(b):
        dummy = out_hbm.at[pl.ds(0, CHUNK)]
        pltpu.make_async_copy(dummy, rows_s[b], sem_s[b]).wait()
        pltpu.make_async_copy(dummy, rows_d[b], sem_d[b]).wait()

    def fire_out(g, b):
        pltpu.async_copy(rows_s[b], out_hbm.at[pl.ds(w_base + g * CHUNK, CHUNK)],
                         sem_o[b])

    def drain_out(b):
        dummy = out_hbm.at[pl.ds(0, CHUNK)]
        pltpu.make_async_copy(dummy, rows_s[b], sem_o[b]).wait()

    def mult(b):
        def mul_body(i, _):
            for j in range(D // 16):
                sl = (i, pl.ds(j * 16, 16))
                rows_s[b][sl] = rows_s[b][sl] * rows_d[b][sl]
            return 0

        lax.fori_loop(0, CHUNK, mul_body, 0)

    fire_gather(0, 0)
    fire_gather(1, 1)

    main_iters = (N_CHUNKS - 2) // NBUF

    def steady(g2, _):
        for k in range(NBUF):
            g = g2 * NBUF + k
            drain_gather(k)
            mult(k)
            fire_out(g, k)
            b2 = (k + 2) % NBUF
            if k == 0:
                @pl.when(g2 >= 1)
                def _():
                    drain_out(b2)
            else:
                drain_out(b2)
            fire_gather(g + 2, b2)
        return 0

    lax.fori_loop(0, main_iters, steady, 0)

    for g, b in ((N_CHUNKS - 2, 0), (N_CHUNKS - 1, 1)):
        drain_gather(b)
        mult(b)
        fire_out(g, b)
    drain_out(2)
    drain_out(0)
    drain_out(1)


@jax.jit
def _edge_attn(node_attn, src, dst):
    mesh = plsc.VectorSubcoreMesh(core_axis_name="c", subcore_axis_name="s")
    return pl.kernel(
        _edge_attn_body,
        mesh=mesh,
        out_type=jax.ShapeDtypeStruct((N_EDGES, D), jnp.float32),
        scratch_types=[
            pltpu.VMEM((E_PER_W,), jnp.int32),
            pltpu.VMEM((E_PER_W,), jnp.int32),
        ] + [pltpu.VMEM((CHUNK, D), jnp.float32)] * 6
          + [pltpu.SemaphoreType.DMA] * 9,
    )(node_attn, src, dst)


def kernel(node_attn, edge_index):
    src = edge_index[0].astype(jnp.int32)
    dst = edge_index[1].astype(jnp.int32)
    return _edge_attn(node_attn, src, dst)
